# Initial kernel scaffold; baseline (speedup 1.0000x reference)
import jax
import jax.numpy as jnp
from jax.experimental import pallas as pl


def kernel(inputs, ind1, w1, lambda1):
    raise NotImplementedError("write your pallas kernel here")


# speedup vs baseline: 3.6451x; 3.6451x over previous
import jax
import jax.numpy as jnp
from jax.experimental import pallas as pl


_K, _H = 4, 32
_NT = 4096


def _gather5_kernel(idx_ref, w_ref, lam_ref, x_ref, o_ref):
    idx = idx_ref[0]
    cols = jax.lax.broadcasted_iota(jnp.int32, (_K, _H, _H), 2)
    onehot = (cols == idx[:, :, None]).astype(jnp.float32)
    w = w_ref[0]
    m = jnp.sum(onehot * w[:, :, None], axis=0) * lam_ref[0, 0]
    a = m
    for _ in range(4):
        a = jax.lax.dot(a, m, precision=jax.lax.Precision.HIGHEST,
                        preferred_element_type=jnp.float32)
    x = x_ref[0]
    o_ref[0] = jax.lax.dot(a, x, precision=jax.lax.Precision.HIGHEST,
                           preferred_element_type=jnp.float32)


def kernel(inputs, ind1, w1, lambda1):
    b, h, w, c = inputs.shape
    k = ind1.shape[0]
    n = w * c
    x = inputs.reshape(b, h, n)
    idx = ind1.reshape(1, k, h)
    wf = w1.reshape(1, k, 1)
    lam = lambda1.reshape(1, 1)
    grid = (b, n // _NT)
    out = pl.pallas_call(
        _gather5_kernel,
        grid=grid,
        in_specs=[
            pl.BlockSpec((1, k, h), lambda i, j: (0, 0, 0)),
            pl.BlockSpec((1, k, 1), lambda i, j: (0, 0, 0)),
            pl.BlockSpec((1, 1), lambda i, j: (0, 0)),
            pl.BlockSpec((1, h, _NT), lambda i, j: (i, 0, j)),
        ],
        out_specs=pl.BlockSpec((1, h, _NT), lambda i, j: (i, 0, j)),
        out_shape=jax.ShapeDtypeStruct((b, h, n), jnp.float32),
    )(idx, wf, lam, x)
    return out.reshape(b, h, w, c)
